# Initial kernel scaffold; baseline (speedup 1.0000x reference)
import jax
import jax.numpy as jnp
from jax.experimental import pallas as pl


def kernel(x, edge_index, W0, a_src0, a_dst0, b0, W1, a_src1, a_dst1, b1, W_mlp, b_mlp):
    raise NotImplementedError("write your pallas kernel here")


# speedup vs baseline: 18.6728x; 18.6728x over previous
import functools

import jax
import jax.numpy as jnp
from jax import lax
from jax.experimental import pallas as pl
from jax.experimental.pallas import tpu as pltpu
from jax.experimental.pallas import tpu_sc as plsc

ALPHA = 0.05
NEG_SLOPE = 0.2

NC = 2
NS = 16
EK = 128


def _leaky(v):
    return jnp.where(v >= 0, v, NEG_SLOPE * v)


def _pre_body(h_ref, w_ref, a2_ref, x_ref, asrc_ref, adst_ref, mx_ref):
    i = pl.program_id(0)
    X = jnp.dot(h_ref[...], w_ref[...], preferred_element_type=jnp.float32)
    x_ref[...] = X
    av = jnp.dot(X, a2_ref[...], preferred_element_type=jnp.float32)
    asrc_ref[...] = av[:, 0:1]
    adst_ref[...] = av[:, 1:2]
    bm = jnp.max(av, axis=0, keepdims=True)

    @pl.when(i == 0)
    def _():
        mx_ref[...] = bm

    @pl.when(i > 0)
    def _():
        mx_ref[...] = jnp.maximum(mx_ref[...], bm)


def _pre_call(h, W, a2, bm=1000):
    n, c = h.shape
    return pl.pallas_call(
        _pre_body,
        grid=(n // bm,),
        in_specs=[pl.BlockSpec((bm, c), lambda i: (i, 0)),
                  pl.BlockSpec((c, c), lambda i: (0, 0)),
                  pl.BlockSpec((c, 2), lambda i: (0, 0))],
        out_specs=[pl.BlockSpec((bm, c), lambda i: (i, 0)),
                   pl.BlockSpec((bm, 1), lambda i: (i, 0)),
                   pl.BlockSpec((bm, 1), lambda i: (i, 0)),
                   pl.BlockSpec((1, 2), lambda i: (0, 0))],
        out_shape=[jax.ShapeDtypeStruct((n, c), jnp.float32),
                   jax.ShapeDtypeStruct((n, 1), jnp.float32),
                   jax.ShapeDtypeStruct((n, 1), jnp.float32),
                   jax.ShapeDtypeStruct((1, 2), jnp.float32)],
    )(h, W, a2)


def _post_body(xin_ref, x_ref, n0_ref, n1_ref, d0_ref, d1_ref,
               asrc_ref, adst_ref, m_ref, b_ref, h_ref):
    w = jnp.exp(_leaky(asrc_ref[...] + adst_ref[...]) - m_ref[...])
    num = n0_ref[...] + n1_ref[...] + w * x_ref[...]
    den = d0_ref[...] + d1_ref[...] + w
    out = num / den + b_ref[...]
    h_ref[...] = ALPHA * xin_ref[...] + (1.0 - ALPHA) * out


def _post_call(xin, X, n0, n1, d0, d1, asrc, adst, m11, b, bm=1000):
    n, c = X.shape
    row = lambda i: (i, 0)
    zero = lambda i: (0, 0)
    return pl.pallas_call(
        _post_body,
        grid=(n // bm,),
        in_specs=[pl.BlockSpec((bm, c), row), pl.BlockSpec((bm, c), row),
                  pl.BlockSpec((bm, c), row), pl.BlockSpec((bm, c), row),
                  pl.BlockSpec((bm, 1), row), pl.BlockSpec((bm, 1), row),
                  pl.BlockSpec((bm, 1), row), pl.BlockSpec((bm, 1), row),
                  pl.BlockSpec((1, 1), zero), pl.BlockSpec((1, c), zero)],
        out_specs=pl.BlockSpec((bm, c), row),
        out_shape=jax.ShapeDtypeStruct((n, c), jnp.float32),
    )(xin, X, n0, n1, d0, d1, asrc, adst, m11, b)


def _final_body(x_ref, h1_ref, h2_ref, w0_ref, w1_ref, w2_ref, b_ref, o_ref):
    o_ref[...] = (
        jnp.dot(x_ref[...], w0_ref[...], preferred_element_type=jnp.float32)
        + jnp.dot(h1_ref[...], w1_ref[...], preferred_element_type=jnp.float32)
        + jnp.dot(h2_ref[...], w2_ref[...], preferred_element_type=jnp.float32)
        + b_ref[...])


def _final_call(x, h1, h2, w0, w1, w2, b, bm=1000):
    n, c = x.shape
    co = w0.shape[1]
    row = lambda i: (i, 0)
    zero = lambda i: (0, 0)
    return pl.pallas_call(
        _final_body,
        grid=(n // bm,),
        in_specs=[pl.BlockSpec((bm, c), row), pl.BlockSpec((bm, c), row),
                  pl.BlockSpec((bm, c), row),
                  pl.BlockSpec((c, co), zero), pl.BlockSpec((c, co), zero),
                  pl.BlockSpec((c, co), zero), pl.BlockSpec((1, co), zero)],
        out_specs=pl.BlockSpec((bm, co), row),
        out_shape=jax.ShapeDtypeStruct((n, co), jnp.float32),
    )(x, h1, h2, w0, w1, w2, b)


def _make_sc_edge(n_pad, c, ept):
    nchunk = ept // EK
    rpt = n_pad // NS
    nzb = rpt // EK
    cg = c // 16

    mesh = plsc.VectorSubcoreMesh(core_axis_name="c", subcore_axis_name="s")

    @functools.partial(
        pl.kernel,
        out_type=[jax.ShapeDtypeStruct((NC, n_pad, c), jnp.float32),
                  jax.ShapeDtypeStruct((NC, n_pad), jnp.float32)],
        mesh=mesh,
        compiler_params=pltpu.CompilerParams(needs_layout_passes=False),
        scratch_types=[
            pltpu.VMEM((n_pad,), jnp.float32),
            pltpu.VMEM((n_pad,), jnp.float32),
            pltpu.VMEM((16,), jnp.float32),
            pltpu.VMEM((EK,), jnp.int32),
            pltpu.VMEM((EK,), jnp.int32),
            pltpu.VMEM((EK,), jnp.float32),
            pltpu.VMEM((EK, c), jnp.float32),
            pltpu.VMEM_SHARED((n_pad, c), jnp.float32),
            pltpu.VMEM_SHARED((n_pad,), jnp.float32),
            pltpu.SemaphoreType.DMA,
        ],
    )
    def sc_edge(x_h, asrc_h, adst_h, src_h, dst_h, m_h, num_o, den_o,
                asrc_v, adst_v, m_v, srcv, dstv, wv, rows,
                num_sh, den_sh, sem):
        ci = lax.axis_index("c")
        si = lax.axis_index("s")
        tid = ci * NS + si

        pltpu.sync_copy(asrc_h, asrc_v)
        pltpu.sync_copy(adst_h, adst_v)
        pltpu.sync_copy(m_h, m_v)

        def zrow(k, _):
            for g in range(cg):
                rows[k, pl.ds(g * 16, 16)] = jnp.zeros((16,), jnp.float32)
            return 0
        lax.fori_loop(0, EK, zrow, 0)

        def zacc(j, _):
            r0 = si * rpt + j * EK
            pltpu.sync_copy(rows, num_sh.at[pl.ds(r0, EK)])
            pltpu.sync_copy(rows.at[0], den_sh.at[pl.ds(r0, EK)])
            return 0
        lax.fori_loop(0, nzb, zacc, 0)
        plsc.subcore_barrier()

        mvec = m_v[...]
        ebase = tid * ept

        def chunk(j, _):
            base = ebase + j * EK
            pltpu.sync_copy(src_h.at[pl.ds(base, EK)], srcv)
            pltpu.sync_copy(dst_h.at[pl.ds(base, EK)], dstv)
            gat = pltpu.async_copy(x_h.at[srcv], rows, sem)
            for i in range(EK // 16):
                sidx = srcv[pl.ds(i * 16, 16)]
                didx = dstv[pl.ds(i * 16, 16)]
                e = (plsc.load_gather(asrc_v, [sidx])
                     + plsc.load_gather(adst_v, [didx]))
                wv[pl.ds(i * 16, 16)] = jnp.exp(_leaky(e) - mvec)
            gat.wait()

            def scale(i, _):
                w16 = wv[pl.ds(i * 16, 16)]
                for k in range(16):
                    wk = w16[k]
                    r = i * 16 + k
                    for g in range(cg):
                        rows[r, pl.ds(g * 16, 16)] = rows[r, pl.ds(g * 16, 16)] * wk
                return 0
            lax.fori_loop(0, EK // 16, scale, 0)

            pltpu.sync_copy(rows, num_sh.at[dstv], add=True)
            pltpu.sync_copy(wv, den_sh.at[dstv], add=True)
            return 0
        lax.fori_loop(0, nchunk, chunk, 0)
        plsc.subcore_barrier()

        def copyout(j, _):
            r0 = si * rpt + j * EK
            pltpu.sync_copy(num_sh.at[pl.ds(r0, EK)], num_o.at[ci, pl.ds(r0, EK)])
            pltpu.sync_copy(den_sh.at[pl.ds(r0, EK)], den_o.at[ci, pl.ds(r0, EK)])
            return 0
        lax.fori_loop(0, nzb, copyout, 0)

    return sc_edge


def kernel(x, edge_index, W0, a_src0, a_dst0, b0, W1, a_src1, a_dst1, b1,
           W_mlp, b_mlp):
    n, c = x.shape
    e_total = edge_index.shape[1]
    nw = NC * NS
    blk = NS * EK
    n_pad = ((n + blk - 1) // blk) * blk
    ept = -(-e_total // nw)
    ept = ((ept + EK - 1) // EK) * EK
    pad = nw * ept - e_total

    src_p = jnp.concatenate([edge_index[0], jnp.zeros((pad,), jnp.int32)])
    dst_p = jnp.concatenate(
        [edge_index[1], jnp.full((pad,), n_pad - 1, jnp.int32)])

    sc_edge = _make_sc_edge(n_pad, c, ept)

    def layer(h, W, a_s, a_d, b):
        a2 = jnp.stack([a_s, a_d], axis=1)
        X, asrc, adst, mx = _pre_call(h, W, a2)
        m = _leaky(mx[0, 0] + mx[0, 1])
        asrc_p = jnp.pad(asrc[:, 0], (0, n_pad - n))
        adst_p = jnp.pad(adst[:, 0], (0, n_pad - n))
        m16 = jnp.broadcast_to(m, (16,))
        num, den = sc_edge(X, asrc_p, adst_p, src_p, dst_p, m16)
        h_next = _post_call(
            x, X, num[0, :n], num[1, :n],
            den[0, :n, None], den[1, :n, None],
            asrc, adst, m.reshape(1, 1), b[None, :])
        return h_next, X

    h1, _ = layer(x, W0, a_src0, a_dst0, b0)
    h2, _ = layer(h1, W1, a_src1, a_dst1, b1)
    return _final_call(x, h1, h2, W_mlp[0:c], W_mlp[c:2 * c],
                       W_mlp[2 * c:3 * c], b_mlp[None, :])


# speedup vs baseline: 35.9947x; 1.9277x over previous
import functools

import jax
import jax.numpy as jnp
from jax import lax
from jax.experimental import pallas as pl
from jax.experimental.pallas import tpu as pltpu
from jax.experimental.pallas import tpu_sc as plsc

ALPHA = 0.05
NEG_SLOPE = 0.2

NC = 2
NS = 16
EK = 64
NB = 3


def _leaky(v):
    return jnp.where(v >= 0, v, NEG_SLOPE * v)


def _pre_body(h_ref, w_ref, a2_ref, x_ref, asrc_ref, adst_ref, mx_ref):
    i = pl.program_id(0)
    X = jnp.dot(h_ref[...], w_ref[...], preferred_element_type=jnp.float32)
    x_ref[...] = X
    av = jnp.dot(X, a2_ref[...], preferred_element_type=jnp.float32)
    asrc_ref[...] = av[:, 0:1]
    adst_ref[...] = av[:, 1:2]
    bm = jnp.max(av, axis=0, keepdims=True)

    @pl.when(i == 0)
    def _():
        mx_ref[...] = bm

    @pl.when(i > 0)
    def _():
        mx_ref[...] = jnp.maximum(mx_ref[...], bm)


def _pre_call(h, W, a2, bm=1000):
    n, c = h.shape
    return pl.pallas_call(
        _pre_body,
        grid=(n // bm,),
        in_specs=[pl.BlockSpec((bm, c), lambda i: (i, 0)),
                  pl.BlockSpec((c, c), lambda i: (0, 0)),
                  pl.BlockSpec((c, 2), lambda i: (0, 0))],
        out_specs=[pl.BlockSpec((bm, c), lambda i: (i, 0)),
                   pl.BlockSpec((bm, 1), lambda i: (i, 0)),
                   pl.BlockSpec((bm, 1), lambda i: (i, 0)),
                   pl.BlockSpec((1, 2), lambda i: (0, 0))],
        out_shape=[jax.ShapeDtypeStruct((n, c), jnp.float32),
                   jax.ShapeDtypeStruct((n, 1), jnp.float32),
                   jax.ShapeDtypeStruct((n, 1), jnp.float32),
                   jax.ShapeDtypeStruct((1, 2), jnp.float32)],
    )(h, W, a2)


def _post_body(xin_ref, x_ref, n0_ref, n1_ref, d0_ref, d1_ref,
               asrc_ref, adst_ref, m_ref, b_ref, h_ref):
    w = jnp.exp(_leaky(asrc_ref[...] + adst_ref[...]) - m_ref[...])
    num = n0_ref[...] + n1_ref[...] + w * x_ref[...]
    den = d0_ref[...] + d1_ref[...] + w
    out = num / den + b_ref[...]
    h_ref[...] = ALPHA * xin_ref[...] + (1.0 - ALPHA) * out


def _post_call(xin, X, n0, n1, d0, d1, asrc, adst, m11, b, bm=1000):
    n, c = X.shape
    row = lambda i: (i, 0)
    zero = lambda i: (0, 0)
    return pl.pallas_call(
        _post_body,
        grid=(n // bm,),
        in_specs=[pl.BlockSpec((bm, c), row), pl.BlockSpec((bm, c), row),
                  pl.BlockSpec((bm, c), row), pl.BlockSpec((bm, c), row),
                  pl.BlockSpec((bm, 1), row), pl.BlockSpec((bm, 1), row),
                  pl.BlockSpec((bm, 1), row), pl.BlockSpec((bm, 1), row),
                  pl.BlockSpec((1, 1), zero), pl.BlockSpec((1, c), zero)],
        out_specs=pl.BlockSpec((bm, c), row),
        out_shape=jax.ShapeDtypeStruct((n, c), jnp.float32),
    )(xin, X, n0, n1, d0, d1, asrc, adst, m11, b)


def _final_body(x_ref, h1_ref, h2_ref, w0_ref, w1_ref, w2_ref, b_ref, o_ref):
    o_ref[...] = (
        jnp.dot(x_ref[...], w0_ref[...], preferred_element_type=jnp.float32)
        + jnp.dot(h1_ref[...], w1_ref[...], preferred_element_type=jnp.float32)
        + jnp.dot(h2_ref[...], w2_ref[...], preferred_element_type=jnp.float32)
        + b_ref[...])


def _final_call(x, h1, h2, w0, w1, w2, b, bm=1000):
    n, c = x.shape
    co = w0.shape[1]
    row = lambda i: (i, 0)
    zero = lambda i: (0, 0)
    return pl.pallas_call(
        _final_body,
        grid=(n // bm,),
        in_specs=[pl.BlockSpec((bm, c), row), pl.BlockSpec((bm, c), row),
                  pl.BlockSpec((bm, c), row),
                  pl.BlockSpec((c, co), zero), pl.BlockSpec((c, co), zero),
                  pl.BlockSpec((c, co), zero), pl.BlockSpec((1, co), zero)],
        out_specs=pl.BlockSpec((bm, co), row),
        out_shape=jax.ShapeDtypeStruct((n, co), jnp.float32),
    )(x, h1, h2, w0, w1, w2, b)


def _make_sc_edge(n_pad, c, ept):
    nchunk = ept // EK
    rpt = n_pad // NS
    nzb = rpt // EK
    cg = c // 16

    mesh = plsc.VectorSubcoreMesh(core_axis_name="c", subcore_axis_name="s")

    @functools.partial(
        pl.kernel,
        out_type=[jax.ShapeDtypeStruct((NC, n_pad, c), jnp.float32),
                  jax.ShapeDtypeStruct((NC, n_pad), jnp.float32)],
        mesh=mesh,
        compiler_params=pltpu.CompilerParams(needs_layout_passes=False),
        scratch_types=[
            pltpu.VMEM((n_pad,), jnp.float32),
            pltpu.VMEM((n_pad,), jnp.float32),
            pltpu.VMEM((16,), jnp.float32),
            [pltpu.VMEM((EK,), jnp.int32)] * NB,
            [pltpu.VMEM((EK,), jnp.int32)] * NB,
            [pltpu.VMEM((EK,), jnp.int32)] * NB,
            [pltpu.VMEM((EK,), jnp.float32)] * NB,
            [pltpu.VMEM((EK, c), jnp.float32)] * NB,
            pltpu.VMEM_SHARED((n_pad, c), jnp.float32),
            pltpu.VMEM_SHARED((n_pad,), jnp.float32),
            [pltpu.SemaphoreType.DMA] * NB,
            [pltpu.SemaphoreType.DMA] * NB,
            [pltpu.SemaphoreType.DMA] * NB,
            [pltpu.SemaphoreType.DMA] * NB,
        ],
    )
    def sc_edge(x_h, asrc_h, adst_h, src_h, dst_h, m_h, num_o, den_o,
                asrc_v, adst_v, m_v, srcv, dstv, dsts, wv, rows,
                num_sh, den_sh, sem_i, sem_g, sem_sn, sem_sd):
        ci = lax.axis_index("c")
        si = lax.axis_index("s")
        tid = ci * NS + si

        pltpu.sync_copy(asrc_h, asrc_v)
        pltpu.sync_copy(adst_h, adst_v)
        pltpu.sync_copy(m_h, m_v)

        def zrow(k, _):
            for g in range(cg):
                rows[0][k, pl.ds(g * 16, 16)] = jnp.zeros((16,), jnp.float32)
            return 0
        lax.fori_loop(0, EK, zrow, 0)

        def zacc(j, _):
            pltpu.sync_copy(rows[0], num_sh.at[pl.ds(si * rpt + j * EK, EK)])
            return 0
        lax.fori_loop(0, nzb, zacc, 0)

        def zden(j, _):
            pltpu.sync_copy(rows[0].at[0], den_sh.at[pl.ds(si * rpt + j * c, c)])
            return 0
        lax.fori_loop(0, rpt // c, zden, 0)
        plsc.subcore_barrier()

        mvec = m_v[...]
        ebase = tid * ept

        def issue_idx(i, b):
            base = ebase + i * EK
            pltpu.async_copy(src_h.at[pl.ds(base, EK)], srcv[b], sem_i[b])
            pltpu.async_copy(dst_h.at[pl.ds(base, EK)], dstv[b], sem_i[b])

        def wait_idx(b):
            pltpu.make_async_copy(src_h.at[pl.ds(0, EK)], srcv[b], sem_i[b]).wait()
            pltpu.make_async_copy(dst_h.at[pl.ds(0, EK)], dstv[b], sem_i[b]).wait()

        def drain_scatter(b):
            pltpu.make_async_copy(rows[b], num_sh.at[dsts[b]], sem_sn[b]).wait()
            pltpu.make_async_copy(wv[b], den_sh.at[dsts[b]], sem_sd[b]).wait()

        def step(i, b):
            bn, bnn = (b + 1) % NB, (b + 2) % NB
            if isinstance(i, int):
                cond = lambda p, f: f() if p else None
            else:
                cond = lambda p, f: pl.when(p)(f)
            cond(i + 2 < nchunk, lambda: issue_idx(i + 2, bnn))
            cond(i >= 2, lambda: drain_scatter(bn))
            def start_next():
                wait_idx(bn)
                pltpu.async_copy(x_h.at[srcv[bn]], rows[bn], sem_g[bn])
            cond(i + 1 < nchunk, start_next)
            for q in range(EK // 16):
                sidx = srcv[b][pl.ds(q * 16, 16)]
                didx = dstv[b][pl.ds(q * 16, 16)]
                dsts[b][pl.ds(q * 16, 16)] = didx
                e = (plsc.load_gather(asrc_v, [sidx])
                     + plsc.load_gather(adst_v, [didx]))
                wv[b][pl.ds(q * 16, 16)] = jnp.exp(_leaky(e) - mvec)
            pltpu.make_async_copy(x_h.at[srcv[b]], rows[b], sem_g[b]).wait()
            def scale(q, _):
                w16 = wv[b][pl.ds(q * 16, 16)]
                for k in range(16):
                    wk = w16[k]
                    r = q * 16 + k
                    for g in range(cg):
                        rows[b][r, pl.ds(g * 16, 16)] = (
                            rows[b][r, pl.ds(g * 16, 16)] * wk)
                return 0
            lax.fori_loop(0, EK // 16, scale, 0)
            pltpu.async_copy(rows[b], num_sh.at[dsts[b]], sem_sn[b], add=True)
            pltpu.async_copy(wv[b], den_sh.at[dsts[b]], sem_sd[b], add=True)

        issue_idx(0, 0)
        issue_idx(1, 1)
        wait_idx(0)
        pltpu.async_copy(x_h.at[srcv[0]], rows[0], sem_g[0])

        def triple(g, _):
            i0 = g * NB
            for s in range(NB):
                step(i0 + s, s)
            return 0
        lax.fori_loop(0, nchunk // NB, triple, 0)
        for i in range(NB * (nchunk // NB), nchunk):
            step(i, i % NB)
        for i in (nchunk - 2, nchunk - 1):
            drain_scatter(i % NB)
        plsc.subcore_barrier()

        r0 = si * rpt
        pltpu.sync_copy(num_sh.at[pl.ds(r0, rpt)], num_o.at[ci, pl.ds(r0, rpt)])
        pltpu.sync_copy(den_sh.at[pl.ds(r0, rpt)], den_o.at[ci, pl.ds(r0, rpt)])

    return sc_edge


def kernel(x, edge_index, W0, a_src0, a_dst0, b0, W1, a_src1, a_dst1, b1,
           W_mlp, b_mlp):
    n, c = x.shape
    e_total = edge_index.shape[1]
    nw = NC * NS
    blk = NS * EK
    n_pad = ((n + blk - 1) // blk) * blk
    ept = -(-e_total // nw)
    ept = ((ept + EK - 1) // EK) * EK
    pad = nw * ept - e_total

    src_p = jnp.concatenate([edge_index[0], jnp.zeros((pad,), jnp.int32)])
    dst_p = jnp.concatenate(
        [edge_index[1], jnp.full((pad,), n_pad - 1, jnp.int32)])

    sc_edge = _make_sc_edge(n_pad, c, ept)

    def layer(h, W, a_s, a_d, b):
        a2 = jnp.stack([a_s, a_d], axis=1)
        X, asrc, adst, mx = _pre_call(h, W, a2)
        m = _leaky(mx[0, 0] + mx[0, 1])
        asrc_p = jnp.pad(asrc[:, 0], (0, n_pad - n))
        adst_p = jnp.pad(adst[:, 0], (0, n_pad - n))
        m16 = jnp.broadcast_to(m, (16,))
        num, den = sc_edge(X, asrc_p, adst_p, src_p, dst_p, m16)
        h_next = _post_call(
            x, X, num[0, :n], num[1, :n],
            den[0, :n, None], den[1, :n, None],
            asrc, adst, m.reshape(1, 1), b[None, :])
        return h_next, X

    h1, _ = layer(x, W0, a_src0, a_dst0, b0)
    h2, _ = layer(h1, W1, a_src1, a_dst1, b1)
    return _final_call(x, h1, h2, W_mlp[0:c], W_mlp[c:2 * c],
                       W_mlp[2 * c:3 * c], b_mlp[None, :])


# speedup vs baseline: 38.4655x; 1.0686x over previous
import functools

import jax
import jax.numpy as jnp
from jax import lax
from jax.experimental import pallas as pl
from jax.experimental.pallas import tpu as pltpu
from jax.experimental.pallas import tpu_sc as plsc

ALPHA = 0.05
NEG_SLOPE = 0.2

NC = 2
NS = 16
EK = 64
NB = 3


def _leaky(v):
    return jnp.where(v >= 0, v, NEG_SLOPE * v)


def _pre_body(h_ref, w_ref, a2_ref, x_ref, asrc_ref, adst_ref, mx_ref):
    i = pl.program_id(0)
    X = jnp.dot(h_ref[...], w_ref[...], preferred_element_type=jnp.float32)
    x_ref[...] = X
    av = jnp.dot(X, a2_ref[...], preferred_element_type=jnp.float32)
    asrc_ref[...] = av[:, 0:1]
    adst_ref[...] = av[:, 1:2]
    bm = jnp.max(av, axis=0, keepdims=True)

    @pl.when(i == 0)
    def _():
        mx_ref[...] = bm

    @pl.when(i > 0)
    def _():
        mx_ref[...] = jnp.maximum(mx_ref[...], bm)


def _pre_call(h, W, a2, n_pad, bm=1000):
    n, c = h.shape
    row = lambda i: (i, 0)
    zero = lambda i: (0, 0)
    return pl.pallas_call(
        _pre_body,
        grid=(n // bm,),
        in_specs=[pl.BlockSpec((bm, c), row),
                  pl.BlockSpec((c, c), zero),
                  pl.BlockSpec((c, 2), zero)],
        out_specs=[pl.BlockSpec((bm, c), row),
                   pl.BlockSpec((bm, 1), row),
                   pl.BlockSpec((bm, 1), row),
                   pl.BlockSpec((1, 2), zero)],
        out_shape=[jax.ShapeDtypeStruct((n, c), jnp.float32),
                   jax.ShapeDtypeStruct((n_pad, 1), jnp.float32),
                   jax.ShapeDtypeStruct((n_pad, 1), jnp.float32),
                   jax.ShapeDtypeStruct((1, 2), jnp.float32)],
    )(h, W, a2)


def _softmax_mix(xin_ref, x_ref, nm_ref, d0_ref, d1_ref,
                 asrc_ref, adst_ref, m_ref, b_ref):
    w = jnp.exp(_leaky(asrc_ref[...] + adst_ref[...]) - m_ref[...])
    num = nm_ref[0] + nm_ref[1] + w * x_ref[...]
    den = d0_ref[...] + d1_ref[...] + w
    return ALPHA * xin_ref[...] + (1.0 - ALPHA) * (num / den + b_ref[...])


def _mid_body(xin_ref, x1_ref, nm_ref, d0_ref, d1_ref, asrc_ref, adst_ref,
              m_ref, b_ref, w1_ref, a2_ref,
              h1_ref, x2_ref, asrc2_ref, adst2_ref, mx2_ref):
    i = pl.program_id(0)
    h1 = _softmax_mix(xin_ref, x1_ref, nm_ref, d0_ref, d1_ref,
                      asrc_ref, adst_ref, m_ref, b_ref)
    h1_ref[...] = h1
    X2 = jnp.dot(h1, w1_ref[...], preferred_element_type=jnp.float32)
    x2_ref[...] = X2
    av = jnp.dot(X2, a2_ref[...], preferred_element_type=jnp.float32)
    asrc2_ref[...] = av[:, 0:1]
    adst2_ref[...] = av[:, 1:2]
    bm = jnp.max(av, axis=0, keepdims=True)

    @pl.when(i == 0)
    def _():
        mx2_ref[...] = bm

    @pl.when(i > 0)
    def _():
        mx2_ref[...] = jnp.maximum(mx2_ref[...], bm)


def _mid_call(xin, X1, num, d0, d1, asrc, adst, m11, b, W1, a2, bm=1000):
    n, c = X1.shape
    n_pad = num.shape[1]
    row = lambda i: (i, 0)
    zero = lambda i: (0, 0)
    return pl.pallas_call(
        _mid_body,
        grid=(n // bm,),
        in_specs=[pl.BlockSpec((bm, c), row), pl.BlockSpec((bm, c), row),
                  pl.BlockSpec((NC, bm, c), lambda i: (0, i, 0)),
                  pl.BlockSpec((bm, 1), row), pl.BlockSpec((bm, 1), row),
                  pl.BlockSpec((bm, 1), row), pl.BlockSpec((bm, 1), row),
                  pl.BlockSpec((1, 1), zero), pl.BlockSpec((1, c), zero),
                  pl.BlockSpec((c, c), zero), pl.BlockSpec((c, 2), zero)],
        out_specs=[pl.BlockSpec((bm, c), row), pl.BlockSpec((bm, c), row),
                   pl.BlockSpec((bm, 1), row), pl.BlockSpec((bm, 1), row),
                   pl.BlockSpec((1, 2), zero)],
        out_shape=[jax.ShapeDtypeStruct((n, c), jnp.float32),
                   jax.ShapeDtypeStruct((n, c), jnp.float32),
                   jax.ShapeDtypeStruct((n_pad, 1), jnp.float32),
                   jax.ShapeDtypeStruct((n_pad, 1), jnp.float32),
                   jax.ShapeDtypeStruct((1, 2), jnp.float32)],
    )(xin, X1, num, d0, d1, asrc, adst, m11, b, W1, a2)


def _fin_body(xin_ref, h1_ref, x2_ref, nm_ref, d0_ref, d1_ref,
              asrc_ref, adst_ref, m_ref, b_ref,
              w0_ref, w1_ref, w2_ref, bm_ref, o_ref):
    h2 = _softmax_mix(xin_ref, x2_ref, nm_ref, d0_ref, d1_ref,
                      asrc_ref, adst_ref, m_ref, b_ref)
    o_ref[...] = (
        jnp.dot(xin_ref[...], w0_ref[...], preferred_element_type=jnp.float32)
        + jnp.dot(h1_ref[...], w1_ref[...], preferred_element_type=jnp.float32)
        + jnp.dot(h2, w2_ref[...], preferred_element_type=jnp.float32)
        + bm_ref[...])


def _fin_call(xin, h1, X2, num, d0, d1, asrc, adst, m11, b,
              w0, w1, w2, b_mlp, bm=1000):
    n, c = X2.shape
    co = w0.shape[1]
    row = lambda i: (i, 0)
    zero = lambda i: (0, 0)
    return pl.pallas_call(
        _fin_body,
        grid=(n // bm,),
        in_specs=[pl.BlockSpec((bm, c), row), pl.BlockSpec((bm, c), row),
                  pl.BlockSpec((bm, c), row),
                  pl.BlockSpec((NC, bm, c), lambda i: (0, i, 0)),
                  pl.BlockSpec((bm, 1), row), pl.BlockSpec((bm, 1), row),
                  pl.BlockSpec((bm, 1), row), pl.BlockSpec((bm, 1), row),
                  pl.BlockSpec((1, 1), zero), pl.BlockSpec((1, c), zero),
                  pl.BlockSpec((c, co), zero), pl.BlockSpec((c, co), zero),
                  pl.BlockSpec((c, co), zero), pl.BlockSpec((1, co), zero)],
        out_specs=pl.BlockSpec((bm, co), row),
        out_shape=jax.ShapeDtypeStruct((n, co), jnp.float32),
    )(xin, h1, X2, num, d0, d1, asrc, adst, m11, b, w0, w1, w2, b_mlp)


def _make_sc_edge(n_pad, c, ept):
    nchunk = ept // EK
    rpt = n_pad // NS
    nzb = rpt // EK
    cg = c // 16

    mesh = plsc.VectorSubcoreMesh(core_axis_name="c", subcore_axis_name="s")

    @functools.partial(
        pl.kernel,
        out_type=[jax.ShapeDtypeStruct((NC, n_pad, c), jnp.float32),
                  jax.ShapeDtypeStruct((NC, n_pad), jnp.float32)],
        mesh=mesh,
        compiler_params=pltpu.CompilerParams(needs_layout_passes=False),
        scratch_types=[
            pltpu.VMEM((n_pad,), jnp.float32),
            pltpu.VMEM((n_pad,), jnp.float32),
            pltpu.VMEM((16,), jnp.float32),
            [pltpu.VMEM((EK,), jnp.int32)] * NB,
            [pltpu.VMEM((EK,), jnp.int32)] * NB,
            [pltpu.VMEM((EK,), jnp.int32)] * NB,
            [pltpu.VMEM((EK,), jnp.float32)] * NB,
            [pltpu.VMEM((EK, c), jnp.float32)] * NB,
            pltpu.VMEM_SHARED((n_pad, c), jnp.float32),
            pltpu.VMEM_SHARED((n_pad,), jnp.float32),
            [pltpu.SemaphoreType.DMA] * NB,
            [pltpu.SemaphoreType.DMA] * NB,
            [pltpu.SemaphoreType.DMA] * NB,
            [pltpu.SemaphoreType.DMA] * NB,
        ],
    )
    def sc_edge(x_h, asrc_h, adst_h, src_h, dst_h, m_h, num_o, den_o,
                asrc_v, adst_v, m_v, srcv, dstv, dsts, wv, rows,
                num_sh, den_sh, sem_i, sem_g, sem_sn, sem_sd):
        ci = lax.axis_index("c")
        si = lax.axis_index("s")
        tid = ci * NS + si

        pltpu.sync_copy(asrc_h, asrc_v)
        pltpu.sync_copy(adst_h, adst_v)
        pltpu.sync_copy(m_h, m_v)

        def zrow(k, _):
            for g in range(cg):
                rows[0][k, pl.ds(g * 16, 16)] = jnp.zeros((16,), jnp.float32)
            return 0
        lax.fori_loop(0, EK, zrow, 0)

        def zacc(j, _):
            pltpu.sync_copy(rows[0], num_sh.at[pl.ds(si * rpt + j * EK, EK)])
            return 0
        lax.fori_loop(0, nzb, zacc, 0)

        def zden(j, _):
            pltpu.sync_copy(rows[0].at[0], den_sh.at[pl.ds(si * rpt + j * c, c)])
            return 0
        lax.fori_loop(0, rpt // c, zden, 0)
        plsc.subcore_barrier()

        mvec = m_v[...]
        ebase = tid * ept

        def issue_idx(i, b):
            base = ebase + i * EK
            pltpu.async_copy(src_h.at[pl.ds(base, EK)], srcv[b], sem_i[b])
            pltpu.async_copy(dst_h.at[pl.ds(base, EK)], dstv[b], sem_i[b])

        def wait_idx(b):
            pltpu.make_async_copy(src_h.at[pl.ds(0, EK)], srcv[b], sem_i[b]).wait()
            pltpu.make_async_copy(dst_h.at[pl.ds(0, EK)], dstv[b], sem_i[b]).wait()

        def drain_scatter(b):
            pltpu.make_async_copy(rows[b], num_sh.at[dsts[b]], sem_sn[b]).wait()
            pltpu.make_async_copy(wv[b], den_sh.at[dsts[b]], sem_sd[b]).wait()

        def step(i, b):
            bn, bnn = (b + 1) % NB, (b + 2) % NB
            if isinstance(i, int):
                cond = lambda p, f: f() if p else None
            else:
                cond = lambda p, f: pl.when(p)(f)
            cond(i + 2 < nchunk, lambda: issue_idx(i + 2, bnn))
            cond(i >= 2, lambda: drain_scatter(bn))
            def start_next():
                wait_idx(bn)
                pltpu.async_copy(x_h.at[srcv[bn]], rows[bn], sem_g[bn])
            cond(i + 1 < nchunk, start_next)
            for q in range(EK // 16):
                sidx = srcv[b][pl.ds(q * 16, 16)]
                didx = dstv[b][pl.ds(q * 16, 16)]
                dsts[b][pl.ds(q * 16, 16)] = didx
                e = (plsc.load_gather(asrc_v, [sidx])
                     + plsc.load_gather(adst_v, [didx]))
                wv[b][pl.ds(q * 16, 16)] = jnp.exp(_leaky(e) - mvec)
            pltpu.make_async_copy(x_h.at[srcv[b]], rows[b], sem_g[b]).wait()
            def scale(q, _):
                w16 = wv[b][pl.ds(q * 16, 16)]
                for k in range(16):
                    wk = w16[k]
                    r = q * 16 + k
                    for g in range(cg):
                        rows[b][r, pl.ds(g * 16, 16)] = (
                            rows[b][r, pl.ds(g * 16, 16)] * wk)
                return 0
            lax.fori_loop(0, EK // 16, scale, 0)
            pltpu.async_copy(rows[b], num_sh.at[dsts[b]], sem_sn[b], add=True)
            pltpu.async_copy(wv[b], den_sh.at[dsts[b]], sem_sd[b], add=True)

        issue_idx(0, 0)
        issue_idx(1, 1)
        wait_idx(0)
        pltpu.async_copy(x_h.at[srcv[0]], rows[0], sem_g[0])

        def triple(g, _):
            i0 = g * NB
            for s in range(NB):
                step(i0 + s, s)
            return 0
        lax.fori_loop(0, nchunk // NB, triple, 0)
        for i in range(NB * (nchunk // NB), nchunk):
            step(i, i % NB)
        for i in (nchunk - 2, nchunk - 1):
            drain_scatter(i % NB)
        plsc.subcore_barrier()

        r0 = si * rpt
        pltpu.sync_copy(num_sh.at[pl.ds(r0, rpt)], num_o.at[ci, pl.ds(r0, rpt)])
        pltpu.sync_copy(den_sh.at[pl.ds(r0, rpt)], den_o.at[ci, pl.ds(r0, rpt)])

    return sc_edge


def kernel(x, edge_index, W0, a_src0, a_dst0, b0, W1, a_src1, a_dst1, b1,
           W_mlp, b_mlp):
    n, c = x.shape
    e_total = edge_index.shape[1]
    nw = NC * NS
    blk = NS * EK
    n_pad = ((n + blk - 1) // blk) * blk
    ept = -(-e_total // nw)
    ept = ((ept + EK - 1) // EK) * EK
    pad = nw * ept - e_total

    src_p = jnp.concatenate([edge_index[0], jnp.zeros((pad,), jnp.int32)])
    dst_p = jnp.concatenate(
        [edge_index[1], jnp.full((pad,), n_pad - 1, jnp.int32)])

    sc_edge = _make_sc_edge(n_pad, c, ept)

    def run_sc(X, asrc, adst, mx):
        m = _leaky(mx[0, 0] + mx[0, 1])
        num, den = sc_edge(X, asrc.reshape(n_pad), adst.reshape(n_pad),
                           src_p, dst_p, jnp.broadcast_to(m, (16,)))
        return num, den[0, :n, None], den[1, :n, None], m.reshape(1, 1)

    a2_0 = jnp.stack([a_src0, a_dst0], axis=1)
    a2_1 = jnp.stack([a_src1, a_dst1], axis=1)

    X1, asrc1, adst1, mx1 = _pre_call(x, W0, a2_0, n_pad)
    num1, d10, d11, m1 = run_sc(X1, asrc1, adst1, mx1)
    h1, X2, asrc2, adst2, mx2 = _mid_call(
        x, X1, num1, d10, d11, asrc1, adst1, m1, b0[None, :], W1, a2_1)
    num2, d20, d21, m2 = run_sc(X2, asrc2, adst2, mx2)
    return _fin_call(x, h1, X2, num2, d20, d21, asrc2, adst2, m2,
                     b1[None, :], W_mlp[0:c], W_mlp[c:2 * c],
                     W_mlp[2 * c:3 * c], b_mlp[None, :])
